# baseline (speedup 1.0000x reference)
import functools

import jax
import jax.numpy as jnp
from jax import lax
from jax.experimental import pallas as pl
from jax.experimental.pallas import tpu as pltpu
from jax.experimental.pallas import tpu_sc as plsc

N = 10000
E = 320000
IN_CH = 128
H1 = 256
H2 = 128
NC, NS = 2, 16
NW = NC * NS
CH = 128
EPT = 10240
E_PAD = EPT * NW
PAIRS = EPT // CH // 2
ACC_N = 10240
RPT = ACC_N // NS
DCH = 80
DNCH = E // NW // DCH
DEGW = 128
SLOPE = (1.0 / 8 + 1.0 / 3) / 2.0

_MESH = plsc.VectorSubcoreMesh(
    core_axis_name="c", subcore_axis_name="s", num_cores=NC, num_subcores=NS
)


@functools.partial(
    pl.kernel,
    out_type=jax.ShapeDtypeStruct((NC, ACC_N, DEGW), jnp.float32),
    mesh=_MESH,
    scratch_types=[
        pltpu.VMEM((DCH,), jnp.int32),
        pltpu.VMEM((DCH, DEGW), jnp.float32),
        pltpu.VMEM_SHARED((ACC_N, DEGW), jnp.float32),
    ],
)
def _deg_kernel(dst_hbm, ones_hbm, zero_hbm, out_hbm, didx, ones_v, acc):
    c = lax.axis_index("c")
    s = lax.axis_index("s")
    base0 = (c * NS + s) * (E // NW)
    r0 = s * RPT
    pltpu.sync_copy(ones_hbm, ones_v)
    pltpu.sync_copy(zero_hbm.at[pl.ds(r0, RPT)], acc.at[pl.ds(r0, RPT)])
    plsc.subcore_barrier()

    def body(i, carry):
        b = base0 + i * DCH
        pltpu.sync_copy(dst_hbm.at[pl.ds(b, DCH)], didx)
        pltpu.sync_copy(ones_v, acc.at[didx], add=True)
        return carry

    lax.fori_loop(0, DNCH, body, 0)
    plsc.subcore_barrier()
    pltpu.sync_copy(acc.at[pl.ds(r0, RPT)], out_hbm.at[c, pl.ds(r0, RPT)])


@functools.partial(
    pl.kernel,
    out_type=jax.ShapeDtypeStruct((NC, ACC_N, H2), jnp.float32),
    mesh=_MESH,
    scratch_types=[
        pltpu.VMEM((CH,), jnp.int32),
        pltpu.VMEM((CH,), jnp.int32),
        pltpu.VMEM((CH,), jnp.int32),
        pltpu.VMEM((CH,), jnp.int32),
        pltpu.VMEM((CH, H2), jnp.float32),
        pltpu.VMEM((CH, H2), jnp.float32),
        pltpu.VMEM_SHARED((ACC_N, H2), jnp.float32),
        pltpu.SemaphoreType.DMA,
        pltpu.SemaphoreType.DMA,
        pltpu.SemaphoreType.DMA,
        pltpu.SemaphoreType.DMA,
    ],
)
def _spmm_kernel(table_hbm, src_hbm, dst_hbm, zero_hbm, out_hbm,
                 sidx0, sidx1, didx0, didx1, rows0, rows1, acc,
                 gsem0, gsem1, ssem0, ssem1):
    c = lax.axis_index("c")
    s = lax.axis_index("s")
    base0 = (c * NS + s) * EPT
    r0 = s * RPT
    pltpu.sync_copy(zero_hbm.at[pl.ds(r0, RPT)], acc.at[pl.ds(r0, RPT)])
    plsc.subcore_barrier()

    def load_idx(b, si, di):
        pltpu.sync_copy(src_hbm.at[pl.ds(b, CH)], si)
        pltpu.sync_copy(dst_hbm.at[pl.ds(b, CH)], di)

    load_idx(base0, sidx0, didx0)
    pltpu.async_copy(table_hbm.at[sidx0], rows0, gsem0)
    load_idx(base0 + CH, sidx1, didx1)
    pltpu.async_copy(table_hbm.at[sidx1], rows1, gsem1)

    def body(j, carry):
        b2 = base0 + (2 * j + 2) * CH
        pltpu.make_async_copy(table_hbm.at[sidx0], rows0, gsem0).wait()
        pltpu.async_copy(rows0, acc.at[didx0], ssem0, add=True)
        pltpu.make_async_copy(table_hbm.at[sidx1], rows1, gsem1).wait()
        pltpu.async_copy(rows1, acc.at[didx1], ssem1, add=True)
        pltpu.make_async_copy(rows0, acc.at[didx0], ssem0).wait()
        load_idx(b2, sidx0, didx0)
        pltpu.async_copy(table_hbm.at[sidx0], rows0, gsem0)
        pltpu.make_async_copy(rows1, acc.at[didx1], ssem1).wait()
        load_idx(b2 + CH, sidx1, didx1)
        pltpu.async_copy(table_hbm.at[sidx1], rows1, gsem1)
        return carry

    lax.fori_loop(0, PAIRS - 1, body, 0)
    pltpu.make_async_copy(table_hbm.at[sidx0], rows0, gsem0).wait()
    pltpu.async_copy(rows0, acc.at[didx0], ssem0, add=True)
    pltpu.make_async_copy(table_hbm.at[sidx1], rows1, gsem1).wait()
    pltpu.async_copy(rows1, acc.at[didx1], ssem1, add=True)
    pltpu.make_async_copy(rows0, acc.at[didx0], ssem0).wait()
    pltpu.make_async_copy(rows1, acc.at[didx1], ssem1).wait()
    plsc.subcore_barrier()
    pltpu.sync_copy(acc.at[pl.ds(r0, RPT)], out_hbm.at[c, pl.ds(r0, RPT)])


_RB = 2000


def _rsqrt_deg(degp_ref):
    deg = degp_ref[0, :, 0:1] + degp_ref[1, :, 0:1] + 1.0
    return lax.rsqrt(deg)


def _tc1_body(x_ref, w1_ref, degp_ref, h1a_ref, h1b_ref):
    xh = jnp.dot(x_ref[...], w1_ref[...], preferred_element_type=jnp.float32)
    d = _rsqrt_deg(degp_ref)
    h = xh * d
    h1a_ref[...] = h[:, :H2]
    h1b_ref[...] = h[:, H2:]


def _tc2_body(a1a_ref, a1b_ref, h1a_ref, h1b_ref, degp_ref,
              w2a_ref, w2b_ref, b1_ref, h2_ref):
    d = _rsqrt_deg(degp_ref)
    ua = d * (a1a_ref[0] + a1a_ref[1] + h1a_ref[...]) + b1_ref[:, :H2]
    ub = d * (a1b_ref[0] + a1b_ref[1] + h1b_ref[...]) + b1_ref[:, H2:]
    ra = jnp.where(ua >= 0, ua, ua * SLOPE)
    rb = jnp.where(ub >= 0, ub, ub * SLOPE)
    xh2 = (jnp.dot(ra, w2a_ref[...], preferred_element_type=jnp.float32)
           + jnp.dot(rb, w2b_ref[...], preferred_element_type=jnp.float32))
    h2_ref[...] = xh2 * d


def _tc3_body(a2_ref, h2_ref, degp_ref, b2_ref, z_ref):
    d = _rsqrt_deg(degp_ref)
    z_ref[...] = d * (a2_ref[0] + a2_ref[1] + h2_ref[...]) + b2_ref[...]


def _row_spec(w):
    return pl.BlockSpec((_RB, w), lambda i: (i, 0))


def _part_spec(w):
    return pl.BlockSpec((NC, _RB, w), lambda i: (0, i, 0))


_DEG_SPEC = pl.BlockSpec((NC, _RB, DEGW), lambda i: (0, i, 0))
_GRID = (N // _RB,)

_tc1 = pl.pallas_call(
    _tc1_body,
    grid=_GRID,
    in_specs=[
        _row_spec(IN_CH),
        pl.BlockSpec((IN_CH, H1), lambda i: (0, 0)),
        _DEG_SPEC,
    ],
    out_specs=[_row_spec(H2), _row_spec(H2)],
    out_shape=[
        jax.ShapeDtypeStruct((N, H2), jnp.float32),
        jax.ShapeDtypeStruct((N, H2), jnp.float32),
    ],
)

_tc2 = pl.pallas_call(
    _tc2_body,
    grid=_GRID,
    in_specs=[
        _part_spec(H2),
        _part_spec(H2),
        _row_spec(H2),
        _row_spec(H2),
        _DEG_SPEC,
        pl.BlockSpec((H2, H2), lambda i: (0, 0)),
        pl.BlockSpec((H2, H2), lambda i: (0, 0)),
        pl.BlockSpec((1, H1), lambda i: (0, 0)),
    ],
    out_specs=_row_spec(H2),
    out_shape=jax.ShapeDtypeStruct((N, H2), jnp.float32),
)

_tc3 = pl.pallas_call(
    _tc3_body,
    grid=_GRID,
    in_specs=[
        _part_spec(H2),
        _row_spec(H2),
        _DEG_SPEC,
        pl.BlockSpec((1, H2), lambda i: (0, 0)),
    ],
    out_specs=_row_spec(H2),
    out_shape=jax.ShapeDtypeStruct((N, H2), jnp.float32),
)


def kernel(x, edge_index, W1, b1, W2, b2):
    src = edge_index[0].astype(jnp.int32)
    dst = edge_index[1].astype(jnp.int32)
    pad = E_PAD - E
    src_p = jnp.concatenate([src, jnp.zeros((pad,), jnp.int32)])
    dst_p = jnp.concatenate([dst, jnp.full((pad,), ACC_N - 1, jnp.int32)])
    ones_rows = jnp.ones((DCH, DEGW), jnp.float32)
    zdeg = jnp.zeros((ACC_N, DEGW), jnp.float32)
    zacc = jnp.zeros((ACC_N, H2), jnp.float32)

    degp = _deg_kernel(dst, ones_rows, zdeg)
    h1a, h1b = _tc1(x, W1, degp)
    agg1a = _spmm_kernel(h1a, src_p, dst_p, zacc)
    agg1b = _spmm_kernel(h1b, src_p, dst_p, zacc)
    h2 = _tc2(agg1a, agg1b, h1a, h1b, degp,
              W2[:H2], W2[H2:], b1.reshape(1, H1))
    agg2 = _spmm_kernel(h2, src_p, dst_p, zacc)
    z = _tc3(agg2, h2, degp, b2.reshape(1, H2))
    return z
